# baseline (speedup 1.0000x reference)
import functools

import jax
import jax.numpy as jnp
from jax import lax
from jax.experimental import pallas as pl
from jax.experimental.pallas import tpu as pltpu
from jax.experimental.pallas import tpu_sc as plsc

_N = 10000
_E = 320000
_D = 128

_NC = 2
_NS = 16
_NW = _NC * _NS

_C = 64
_CHUNKS = 160
_TPT = _C * _CHUNKS
_EPAD = _TPT * _NW
_EROWS = _EPAD // _C
_CPT = _EROWS // _NW
_SCPT = 40
_CH0 = 320
_CH1 = 0

_NROWS = 10112
_RPT = _NROWS // _NS
_DROWS = 80

_mesh = plsc.VectorSubcoreMesh(core_axis_name="c", subcore_axis_name="s")


@functools.partial(
    pl.kernel,
    mesh=_mesh,
    out_type=jax.ShapeDtypeStruct((_NC * _NROWS, _D), jnp.float32),
    scratch_types=[
        pltpu.VMEM((_CPT, _C), jnp.int32),
        pltpu.VMEM((_C, _D), jnp.float32),
        pltpu.VMEM_SHARED((_NROWS, _D), jnp.float32),
        pltpu.SemaphoreType.DMA,
    ],
)
def _deg_kernel(dst_hbm, out_hbm, didx, ones_v, acc, sem):
    cid = lax.axis_index("c")
    sid = lax.axis_index("s")
    wid = cid * _NS + sid

    def _zero(i, _):
        for c in range(_D // 16):
            ones_v[i, pl.ds(c * 16, 16)] = jnp.zeros((16,), jnp.float32)
        return 0

    lax.fori_loop(0, _C, _zero, 0)

    r0 = sid * _RPT
    for j in range(_RPT // _C):
        pltpu.sync_copy(ones_v, acc.at[pl.ds(r0 + j * _C, _C)])
    pltpu.sync_copy(ones_v.at[pl.ds(0, _RPT % _C)],
                    acc.at[pl.ds(r0 + (_RPT // _C) * _C, _RPT % _C)])

    def _fill(i, _):
        for c in range(_D // 16):
            ones_v[i, pl.ds(c * 16, 16)] = jnp.full((16,), 1.0, jnp.float32)
        return 0

    lax.fori_loop(0, _C, _fill, 0)

    pltpu.sync_copy(dst_hbm.at[pl.ds(wid * _CPT, _CPT)], didx)
    plsc.subcore_barrier()

    def _body(i, _):
        pltpu.sync_copy(ones_v, acc.at[didx.at[i]], add=True)
        return 0

    lax.fori_loop(0, _CHUNKS, _body, 0)
    plsc.subcore_barrier()

    pltpu.sync_copy(acc.at[pl.ds(r0, _RPT)],
                    out_hbm.at[pl.ds(cid * _NROWS + r0, _RPT)])


@functools.partial(
    pl.kernel,
    mesh=_mesh,
    out_type=jax.ShapeDtypeStruct((_NC * _NROWS, _D), jnp.float32),
    scratch_types=[
        pltpu.VMEM((_SCPT, _C), jnp.int32),
        pltpu.VMEM((_SCPT, _C), jnp.int32),
        pltpu.VMEM((_C, _D), jnp.float32),
        pltpu.VMEM((_C, _D), jnp.float32),
        pltpu.VMEM((_C, _D), jnp.float32),
        pltpu.VMEM((_C, _D), jnp.float32),
        pltpu.VMEM_SHARED((_NROWS, _D), jnp.float32),
        pltpu.SemaphoreType.DMA,
        pltpu.SemaphoreType.DMA,
        pltpu.SemaphoreType.DMA,
        pltpu.SemaphoreType.DMA,
        pltpu.SemaphoreType.DMA,
        pltpu.SemaphoreType.DMA,
        pltpu.SemaphoreType.DMA,
        pltpu.SemaphoreType.DMA,
    ],
)
def _agg_kernel(src_hbm, dst_hbm, h_hbm, out_hbm, sidx, didx,
                rows0, rows1, rows2, rows3, acc,
                g0, g1, g2, g3, s0, s1, s2, s3):
    cid = lax.axis_index("c")
    sid = lax.axis_index("s")
    wid = cid * _NS + sid
    bufs = [(rows0, g0, s0), (rows1, g1, s1), (rows2, g2, s2), (rows3, g3, s3)]

    def _zero(i, _):
        for c in range(_D // 16):
            rows0[i, pl.ds(c * 16, 16)] = jnp.zeros((16,), jnp.float32)
        return 0

    lax.fori_loop(0, _C, _zero, 0)

    r0 = sid * _RPT
    for j in range(_RPT // _C):
        pltpu.sync_copy(rows0, acc.at[pl.ds(r0 + j * _C, _C)])
    pltpu.sync_copy(rows0.at[pl.ds(0, _RPT % _C)],
                    acc.at[pl.ds(r0 + (_RPT // _C) * _C, _RPT % _C)])
    plsc.subcore_barrier()

    def _step(i, rb, gb, sb, rf, gf, sf):
        pltpu.make_async_copy(h_hbm.at[pl.ds(0, _C)], rb, gb).wait()

        @pl.when(i >= 2)
        def _():
            pltpu.make_async_copy(rf, acc.at[pl.ds(0, _C)], sf).wait()

        @pl.when(i + 2 < _SCPT)
        def _():
            pltpu.async_copy(h_hbm.at[sidx.at[i + 2]], rf, gf)

        pltpu.async_copy(rb, acc.at[didx.at[i]], sb, add=True)

    tile_base = jnp.where(cid == 0, sid * _CH0, _NS * _CH0 + sid * _CH1)
    nstages = jnp.where(cid == 0, _CH0 // _SCPT, _CH1 // _SCPT)

    def _stage(stage, _):
        base = tile_base + stage * _SCPT
        pltpu.sync_copy(src_hbm.at[pl.ds(base, _SCPT)], sidx)
        pltpu.sync_copy(dst_hbm.at[pl.ds(base, _SCPT)], didx)
        pltpu.async_copy(h_hbm.at[sidx.at[0]], rows0, g0)
        pltpu.async_copy(h_hbm.at[sidx.at[1]], rows1, g1)

        def _body(i, _):
            for k in range(4):
                @pl.when(lax.rem(i, 4) == k)
                def _(k=k):
                    rb, gb, sb = bufs[k]
                    rf, gf, sf = bufs[(k + 2) % 4]
                    _step(i, rb, gb, sb, rf, gf, sf)

            return 0

        lax.fori_loop(0, _SCPT, _body, 0)
        pltpu.make_async_copy(rows2, acc.at[pl.ds(0, _C)], s2).wait()
        pltpu.make_async_copy(rows3, acc.at[pl.ds(0, _C)], s3).wait()
        return 0

    lax.fori_loop(0, nstages, _stage, 0)

    plsc.subcore_barrier()
    pltpu.sync_copy(acc.at[pl.ds(r0, _RPT)],
                    out_hbm.at[pl.ds(cid * _NROWS + r0, _RPT)])


_BLK = 1000
_GRID = _N // _BLK


def _rowspec(width=_D):
    return pl.BlockSpec((_BLK, width), lambda i: (i, 0))


def _aggspec(width):
    return pl.BlockSpec((_NC, _BLK, width), lambda i: (0, i, 0))


def _wspec(shape):
    return pl.BlockSpec(shape, lambda i: tuple(0 for _ in shape))


def _phase1_body(deg_ref, x_ref, wf_ref, bf_ref, wc_ref, x1_ref, h1_ref,
                 dinv_ref):
    deg = deg_ref[0, :, 0:1] + deg_ref[1, :, 0:1]
    dinv = jnp.where(deg > 0.0, lax.rsqrt(jnp.maximum(deg, 1.0)), 0.0)
    x = x_ref[...]
    x1_ref[...] = jnp.maximum(
        jnp.dot(x, wf_ref[...], preferred_element_type=jnp.float32)
        + bf_ref[...], 0.0)
    h1_ref[...] = dinv * jnp.dot(
        x, wc_ref[...], preferred_element_type=jnp.float32)
    dinv_ref[...] = dinv


_phase1 = pl.pallas_call(
    _phase1_body,
    grid=(_GRID,),
    in_specs=[
        _aggspec(_D),
        _rowspec(),
        _wspec((_D, _D)),
        _wspec((1, _D)),
        _wspec((_D, _D)),
    ],
    out_specs=[_rowspec(), _rowspec(), _rowspec(1)],
    out_shape=[
        jax.ShapeDtypeStruct((_N, _D), jnp.float32),
        jax.ShapeDtypeStruct((_N, _D), jnp.float32),
        jax.ShapeDtypeStruct((_N, 1), jnp.float32),
    ],
)


def _mid_body(agg_ref, dinv_ref, prev_ref, b_in_ref, w1_ref, b1_ref, w2_ref,
              o1_ref, o2_ref):
    agg = agg_ref[0] + agg_ref[1]
    dinv = dinv_ref[...]
    g = jnp.maximum(dinv * agg + b_in_ref[...], 0.0)
    prev = prev_ref[...]
    w1 = w1_ref[...]
    o1 = (jnp.dot(prev, w1[:_D], preferred_element_type=jnp.float32)
          + jnp.dot(g, w1[_D:], preferred_element_type=jnp.float32)
          + b1_ref[...])
    o1_ref[...] = jnp.maximum(o1, 0.0)
    w2 = w2_ref[...]
    o2_ref[...] = dinv * (
        jnp.dot(prev, w2[:_D], preferred_element_type=jnp.float32)
        + jnp.dot(g, w2[_D:], preferred_element_type=jnp.float32))


_mid = pl.pallas_call(
    _mid_body,
    grid=(_GRID,),
    in_specs=[
        _aggspec(_D),
        _rowspec(1),
        _rowspec(),
        _wspec((1, _D)),
        _wspec((2 * _D, _D)),
        _wspec((1, _D)),
        _wspec((2 * _D, _D)),
    ],
    out_specs=[_rowspec(), _rowspec()],
    out_shape=[
        jax.ShapeDtypeStruct((_N, _D), jnp.float32),
        jax.ShapeDtypeStruct((_N, _D), jnp.float32),
    ],
)


def _final_body(agg_ref, dinv_ref, x4_ref, b_ref, out_ref):
    agg = agg_ref[0] + agg_ref[1]
    out_ref[...] = x4_ref[...] + jnp.maximum(
        dinv_ref[...] * agg + b_ref[...], 0.0)


_final = pl.pallas_call(
    _final_body,
    grid=(_GRID,),
    in_specs=[
        _aggspec(_D),
        _rowspec(1),
        _rowspec(),
        _wspec((1, _D)),
    ],
    out_specs=_rowspec(),
    out_shape=jax.ShapeDtypeStruct((_N, _D), jnp.float32),
)


def kernel(x, edge_index, W_fc1, b_fc1, W_c1, b_c1, W_m0, b_m0, W_mc0, b_mc0,
           W_ff, b_ff, W_fc, b_fc):
    src = edge_index[0]
    dst = edge_index[1]
    pad = _EPAD - _E
    src_p = jnp.concatenate([src, jnp.zeros((pad,), jnp.int32)]).reshape(
        _EROWS, _C)
    dst_p = jnp.concatenate([dst, jnp.full((pad,), _N, jnp.int32)]).reshape(
        _EROWS, _C)

    deg = _deg_kernel(dst_p).reshape(_NC, _NROWS, _D)
    x1, h1, dinv = _phase1(deg, x, W_fc1, b_fc1.reshape(1, _D), W_c1)

    agg1 = _agg_kernel(src_p, dst_p, h1).reshape(_NC, _NROWS, _D)
    m1, h2 = _mid(agg1, dinv, x1, b_c1.reshape(1, _D), W_m0,
                  b_m0.reshape(1, _D), W_mc0)

    agg2 = _agg_kernel(src_p, dst_p, h2).reshape(_NC, _NROWS, _D)
    x4, h3 = _mid(agg2, dinv, m1, b_mc0.reshape(1, _D), W_ff,
                  b_ff.reshape(1, _D), W_fc)

    agg3 = _agg_kernel(src_p, dst_p, h3).reshape(_NC, _NROWS, _D)
    return _final(agg3, dinv, x4, b_fc.reshape(1, _D))


# speedup vs baseline: 1.4424x; 1.4424x over previous
import functools

import jax
import jax.numpy as jnp
from jax import lax
from jax.experimental import pallas as pl
from jax.experimental.pallas import tpu as pltpu
from jax.experimental.pallas import tpu_sc as plsc

_N = 10000
_E = 320000
_D = 128

_NC = 2
_NS = 16
_NW = _NC * _NS

_C = 64
_CHUNKS = 160
_TPT = _C * _CHUNKS
_EPAD = _TPT * _NW
_EROWS = _EPAD // _C
_CPT = _EROWS // _NW
_SCPT = 40
_CH0 = 280
_CH1 = 40

_NROWS = 10112
_RPT = _NROWS // _NS
_DROWS = 80

_mesh = plsc.VectorSubcoreMesh(core_axis_name="c", subcore_axis_name="s")


@functools.partial(
    pl.kernel,
    mesh=_mesh,
    out_type=jax.ShapeDtypeStruct((_NC * _NROWS, _D), jnp.float32),
    scratch_types=[
        pltpu.VMEM((_CPT, _C), jnp.int32),
        pltpu.VMEM((_C, _D), jnp.float32),
        pltpu.VMEM_SHARED((_NROWS, _D), jnp.float32),
        pltpu.SemaphoreType.DMA,
    ],
)
def _deg_kernel(dst_hbm, out_hbm, didx, ones_v, acc, sem):
    cid = lax.axis_index("c")
    sid = lax.axis_index("s")
    wid = cid * _NS + sid

    def _zero(i, _):
        for c in range(_D // 16):
            ones_v[i, pl.ds(c * 16, 16)] = jnp.zeros((16,), jnp.float32)
        return 0

    lax.fori_loop(0, _C, _zero, 0)

    r0 = sid * _RPT
    for j in range(_RPT // _C):
        pltpu.sync_copy(ones_v, acc.at[pl.ds(r0 + j * _C, _C)])
    pltpu.sync_copy(ones_v.at[pl.ds(0, _RPT % _C)],
                    acc.at[pl.ds(r0 + (_RPT // _C) * _C, _RPT % _C)])

    def _fill(i, _):
        for c in range(_D // 16):
            ones_v[i, pl.ds(c * 16, 16)] = jnp.full((16,), 1.0, jnp.float32)
        return 0

    lax.fori_loop(0, _C, _fill, 0)

    pltpu.sync_copy(dst_hbm.at[pl.ds(wid * _CPT, _CPT)], didx)
    plsc.subcore_barrier()

    def _body(i, _):
        pltpu.sync_copy(ones_v, acc.at[didx.at[i]], add=True)
        return 0

    lax.fori_loop(0, _CHUNKS, _body, 0)
    plsc.subcore_barrier()

    pltpu.sync_copy(acc.at[pl.ds(r0, _RPT)],
                    out_hbm.at[pl.ds(cid * _NROWS + r0, _RPT)])


@functools.partial(
    pl.kernel,
    mesh=_mesh,
    out_type=jax.ShapeDtypeStruct((_NC * _NROWS, _D), jnp.float32),
    scratch_types=[
        pltpu.VMEM((_SCPT, _C), jnp.int32),
        pltpu.VMEM((_SCPT, _C), jnp.int32),
        pltpu.VMEM((_C, _D), jnp.float32),
        pltpu.VMEM((_C, _D), jnp.float32),
        pltpu.VMEM((_C, _D), jnp.float32),
        pltpu.VMEM((_C, _D), jnp.float32),
        pltpu.VMEM_SHARED((_NROWS, _D), jnp.float32),
        pltpu.SemaphoreType.DMA,
        pltpu.SemaphoreType.DMA,
        pltpu.SemaphoreType.DMA,
        pltpu.SemaphoreType.DMA,
        pltpu.SemaphoreType.DMA,
        pltpu.SemaphoreType.DMA,
        pltpu.SemaphoreType.DMA,
        pltpu.SemaphoreType.DMA,
    ],
)
def _agg_kernel(src_hbm, dst_hbm, h_hbm, out_hbm, sidx, didx,
                rows0, rows1, rows2, rows3, acc,
                g0, g1, g2, g3, s0, s1, s2, s3):
    cid = lax.axis_index("c")
    sid = lax.axis_index("s")
    wid = cid * _NS + sid
    bufs = [(rows0, g0, s0), (rows1, g1, s1), (rows2, g2, s2), (rows3, g3, s3)]

    def _zero(i, _):
        for c in range(_D // 16):
            rows0[i, pl.ds(c * 16, 16)] = jnp.zeros((16,), jnp.float32)
        return 0

    lax.fori_loop(0, _C, _zero, 0)

    r0 = sid * _RPT
    for j in range(_RPT // _C):
        pltpu.sync_copy(rows0, acc.at[pl.ds(r0 + j * _C, _C)])
    pltpu.sync_copy(rows0.at[pl.ds(0, _RPT % _C)],
                    acc.at[pl.ds(r0 + (_RPT // _C) * _C, _RPT % _C)])
    plsc.subcore_barrier()

    def _step(i, rb, gb, sb, rf, gf, sf):
        pltpu.make_async_copy(h_hbm.at[pl.ds(0, _C)], rb, gb).wait()

        @pl.when(i >= 2)
        def _():
            pltpu.make_async_copy(rf, acc.at[pl.ds(0, _C)], sf).wait()

        @pl.when(i + 2 < _SCPT)
        def _():
            pltpu.async_copy(h_hbm.at[sidx.at[i + 2]], rf, gf)

        pltpu.async_copy(rb, acc.at[didx.at[i]], sb, add=True)

    tile_base = jnp.where(cid == 0, sid * _CH0, _NS * _CH0 + sid * _CH1)
    nstages = jnp.where(cid == 0, _CH0 // _SCPT, _CH1 // _SCPT)

    def _stage(stage, _):
        base = tile_base + stage * _SCPT
        pltpu.sync_copy(src_hbm.at[pl.ds(base, _SCPT)], sidx)
        pltpu.sync_copy(dst_hbm.at[pl.ds(base, _SCPT)], didx)
        pltpu.async_copy(h_hbm.at[sidx.at[0]], rows0, g0)
        pltpu.async_copy(h_hbm.at[sidx.at[1]], rows1, g1)

        def _body(i, _):
            for k in range(4):
                @pl.when(lax.rem(i, 4) == k)
                def _(k=k):
                    rb, gb, sb = bufs[k]
                    rf, gf, sf = bufs[(k + 2) % 4]
                    _step(i, rb, gb, sb, rf, gf, sf)

            return 0

        lax.fori_loop(0, _SCPT, _body, 0)
        pltpu.make_async_copy(rows2, acc.at[pl.ds(0, _C)], s2).wait()
        pltpu.make_async_copy(rows3, acc.at[pl.ds(0, _C)], s3).wait()
        return 0

    lax.fori_loop(0, nstages, _stage, 0)

    plsc.subcore_barrier()
    pltpu.sync_copy(acc.at[pl.ds(r0, _RPT)],
                    out_hbm.at[pl.ds(cid * _NROWS + r0, _RPT)])


_BLK = 1000
_GRID = _N // _BLK


def _rowspec(width=_D):
    return pl.BlockSpec((_BLK, width), lambda i: (i, 0))


def _aggspec(width):
    return pl.BlockSpec((_NC, _BLK, width), lambda i: (0, i, 0))


def _wspec(shape):
    return pl.BlockSpec(shape, lambda i: tuple(0 for _ in shape))


def _phase1_body(deg_ref, x_ref, wf_ref, bf_ref, wc_ref, x1_ref, h1_ref,
                 dinv_ref):
    deg = deg_ref[0, :, 0:1] + deg_ref[1, :, 0:1]
    dinv = jnp.where(deg > 0.0, lax.rsqrt(jnp.maximum(deg, 1.0)), 0.0)
    x = x_ref[...]
    x1_ref[...] = jnp.maximum(
        jnp.dot(x, wf_ref[...], preferred_element_type=jnp.float32)
        + bf_ref[...], 0.0)
    h1_ref[...] = dinv * jnp.dot(
        x, wc_ref[...], preferred_element_type=jnp.float32)
    dinv_ref[...] = dinv


_phase1 = pl.pallas_call(
    _phase1_body,
    grid=(_GRID,),
    in_specs=[
        _aggspec(_D),
        _rowspec(),
        _wspec((_D, _D)),
        _wspec((1, _D)),
        _wspec((_D, _D)),
    ],
    out_specs=[_rowspec(), _rowspec(), _rowspec(1)],
    out_shape=[
        jax.ShapeDtypeStruct((_N, _D), jnp.float32),
        jax.ShapeDtypeStruct((_N, _D), jnp.float32),
        jax.ShapeDtypeStruct((_N, 1), jnp.float32),
    ],
)


def _mid_body(agg_ref, dinv_ref, prev_ref, b_in_ref, w1_ref, b1_ref, w2_ref,
              o1_ref, o2_ref):
    agg = agg_ref[0] + agg_ref[1]
    dinv = dinv_ref[...]
    g = jnp.maximum(dinv * agg + b_in_ref[...], 0.0)
    prev = prev_ref[...]
    w1 = w1_ref[...]
    o1 = (jnp.dot(prev, w1[:_D], preferred_element_type=jnp.float32)
          + jnp.dot(g, w1[_D:], preferred_element_type=jnp.float32)
          + b1_ref[...])
    o1_ref[...] = jnp.maximum(o1, 0.0)
    w2 = w2_ref[...]
    o2_ref[...] = dinv * (
        jnp.dot(prev, w2[:_D], preferred_element_type=jnp.float32)
        + jnp.dot(g, w2[_D:], preferred_element_type=jnp.float32))


_mid = pl.pallas_call(
    _mid_body,
    grid=(_GRID,),
    in_specs=[
        _aggspec(_D),
        _rowspec(1),
        _rowspec(),
        _wspec((1, _D)),
        _wspec((2 * _D, _D)),
        _wspec((1, _D)),
        _wspec((2 * _D, _D)),
    ],
    out_specs=[_rowspec(), _rowspec()],
    out_shape=[
        jax.ShapeDtypeStruct((_N, _D), jnp.float32),
        jax.ShapeDtypeStruct((_N, _D), jnp.float32),
    ],
)


def _final_body(agg_ref, dinv_ref, x4_ref, b_ref, out_ref):
    agg = agg_ref[0] + agg_ref[1]
    out_ref[...] = x4_ref[...] + jnp.maximum(
        dinv_ref[...] * agg + b_ref[...], 0.0)


_final = pl.pallas_call(
    _final_body,
    grid=(_GRID,),
    in_specs=[
        _aggspec(_D),
        _rowspec(1),
        _rowspec(),
        _wspec((1, _D)),
    ],
    out_specs=_rowspec(),
    out_shape=jax.ShapeDtypeStruct((_N, _D), jnp.float32),
)


def kernel(x, edge_index, W_fc1, b_fc1, W_c1, b_c1, W_m0, b_m0, W_mc0, b_mc0,
           W_ff, b_ff, W_fc, b_fc):
    src = edge_index[0]
    dst = edge_index[1]
    pad = _EPAD - _E
    src_p = jnp.concatenate([src, jnp.zeros((pad,), jnp.int32)]).reshape(
        _EROWS, _C)
    dst_p = jnp.concatenate([dst, jnp.full((pad,), _N, jnp.int32)]).reshape(
        _EROWS, _C)

    deg = _deg_kernel(dst_p).reshape(_NC, _NROWS, _D)
    x1, h1, dinv = _phase1(deg, x, W_fc1, b_fc1.reshape(1, _D), W_c1)

    agg1 = _agg_kernel(src_p, dst_p, h1).reshape(_NC, _NROWS, _D)
    m1, h2 = _mid(agg1, dinv, x1, b_c1.reshape(1, _D), W_m0,
                  b_m0.reshape(1, _D), W_mc0)

    agg2 = _agg_kernel(src_p, dst_p, h2).reshape(_NC, _NROWS, _D)
    x4, h3 = _mid(agg2, dinv, m1, b_mc0.reshape(1, _D), W_ff,
                  b_ff.reshape(1, _D), W_fc)

    agg3 = _agg_kernel(src_p, dst_p, h3).reshape(_NC, _NROWS, _D)
    return _final(agg3, dinv, x4, b_fc.reshape(1, _D))
